# baseline (speedup 1.0000x reference)
import functools

import jax
import jax.numpy as jnp
from jax import lax
from jax.experimental import pallas as pl
from jax.experimental.pallas import tpu as pltpu
from jax.experimental.pallas import tpu_sc as plsc

N = 1024
K = 128
TAU = 0.1
S = 128
NC = 2
NS = 16
NW = NC * NS
KC = N // NW
RPW = N // NW
KP = N // NW // 2
RC = 256
HR = N // NC
KP2 = N // NS // 2
GB = 8
MASK_HI = -65536
MASK_LO = 65535
NCH = N // RC
ONE_BITS = 0x3F800001


def _tc_body(logits_ref, target_ref, ge_ref, go_ref, a_ref, ld_ref, bp_ref):
    x = logits_ref[...]
    e = jnp.exp(-jnp.abs(x))
    probs = jnp.where(x >= 0.0, 1.0 / (1.0 + e), e / (1.0 + e))
    pb = lax.bitcast_convert_type(probs, jnp.int32)
    ph = (pb >> 15).astype(jnp.int16)
    plo = (pb & 0x7FFF).astype(jnp.int16)

    def rowsum16(x16):
        s = x16[:, :512] + x16[:, 512:]
        s = s[:, :256] + s[:, 256:]
        s = s[:, :128] + s[:, 128:]
        return jnp.sum(s.astype(jnp.int32), axis=1, keepdims=True)

    lo0 = jnp.zeros((N, 1), jnp.int32)
    hi0 = jnp.full((N, 1), 0x7F01, jnp.int32)

    def it_hi(_, lh):
        lo, hi = lh
        mid = (lo + hi) >> 1
        mid16 = mid.astype(jnp.int16)
        cnt = rowsum16((ph >= mid16).astype(jnp.int16))
        ge = cnt >= K
        return jnp.where(ge, mid, lo), jnp.where(ge, hi, mid)

    th32, _ = lax.fori_loop(0, 15, it_hi, (lo0, hi0))
    th = th32.astype(jnp.int16)

    eq = (ph == th).astype(jnp.int16)
    cgt = rowsum16((ph > th).astype(jnp.int16))

    tlo0 = jnp.zeros((N, 1), jnp.int32)
    thi0 = jnp.full((N, 1), 0x8000, jnp.int32)

    def it_lo(_, lh):
        lo, hi = lh
        mid = (lo + hi) >> 1
        mid16 = mid.astype(jnp.int16)
        cnt = cgt + rowsum16(eq * (plo >= mid16).astype(jnp.int16))
        ge = cnt >= K
        return jnp.where(ge, mid, lo), jnp.where(ge, hi, mid)

    tl, _ = lax.fori_loop(0, 15, it_lo, (tlo0, thi0))
    kth = (th32 << 15) | tl
    A = jnp.where(pb >= kth, probs, 0.0)
    a_ref[...] = A

    we = (A * target_ref[...]) * (1.0 / TAU)
    m = jnp.max(we, axis=1, keepdims=True)
    s = jnp.sum(jnp.exp(we - m), axis=1, keepdims=True)
    me = TAU * (m + jnp.log(s))
    ld_ref[...] = jnp.sum((1.0 - me) ** 2, axis=0, keepdims=True) * (1.0 / N)

    lo = lax.bitcast_convert_type(
        ge_ref[...].astype(jnp.bfloat16), jnp.uint16
    ).astype(jnp.int32)
    hi = lax.bitcast_convert_type(
        go_ref[...].astype(jnp.bfloat16), jnp.uint16
    ).astype(jnp.int32)
    bp_ref[...] = lo | (hi << 16)


_tc_call = pl.pallas_call(
    _tc_body,
    out_shape=[
        jax.ShapeDtypeStruct((N, N), jnp.float32),
        jax.ShapeDtypeStruct((1, 1), jnp.float32),
        jax.ShapeDtypeStruct((N // 2, N), jnp.int32),
    ],
)


def _fused_body(a_hbm, bp_hbm, jw_hbm, out_hbm, a_vm, jwx_vm, bp_vm, jw_vm, acc_vm):
    c = lax.axis_index("c")
    s = lax.axis_index("s")
    wid = c * NS + s
    r0 = wid * RPW

    pltpu.sync_copy(a_hbm.at[pl.ds(r0, RPW)], a_vm)

    iota = lax.iota(jnp.int32, 16)
    vone = jnp.full((16,), 1, jnp.int32)
    vzero = jnp.zeros((16,), jnp.int32)

    @plsc.parallel_loop(0, RPW, unroll=2)
    def erow(r):
        base0 = jnp.full((16,), r * S, jnp.int32)
        lim = base0 + S

        def vloop(v, base):
            a = a_vm[r, pl.ds(v * 16, 16)]
            m = a != 0.0
            ones = jnp.where(m, vone, vzero)
            pos = base + (plsc.cumsum(ones) - ones)
            ok = m & (pos < lim)
            wbits = (plsc.bitcast(a, jnp.int32) + 0x8000) & MASK_HI
            plsc.store_scatter(jwx_vm, [pos], (iota + v * 16) | wbits, mask=ok)
            return base + jnp.full((16,), jnp.sum(ones), jnp.int32)

        lax.fori_loop(0, N // 16, vloop, base0)

    pltpu.sync_copy(jwx_vm, jw_hbm.at[pl.ds(r0 * S, RPW * S)])

    plsc.subcore_barrier()

    pltpu.sync_copy(bp_hbm.at[pl.ds(s * KP2, KP2)], bp_vm)

    ksplat = [jnp.full((16,), kp, jnp.int32) for kp in range(KP2)]
    acc = jnp.zeros((16,), jnp.float32)
    for ch in range(HR // RC):
        row_base = c * HR + ch * RC
        pltpu.sync_copy(jw_hbm.at[pl.ds(row_base * S, RC * S)], jw_vm)

        @plsc.parallel_loop(0, RC, carry=acc, unroll=3)
        def row(r, acc):
            ii = jnp.full((16,), row_base + r, jnp.int32)
            for b in range(S // 16 // GB):
                offs = [r * S + (b * GB + t) * 16 for t in range(GB)]
                jws = [jw_vm[pl.ds(o, 16)] for o in offs]
                jvs = [jw & MASK_LO for jw in jws]
                wvs = [plsc.bitcast(jw & MASK_HI, jnp.float32) for jw in jws]
                ds = [jnp.zeros((32,), jnp.bfloat16) for _ in range(GB)]
                for kp in range(KP2):
                    gi = plsc.bitcast(
                        plsc.load_gather(bp_vm, [ksplat[kp], ii]), jnp.bfloat16
                    )
                    for t in range(GB):
                        gj = plsc.bitcast(
                            plsc.load_gather(bp_vm, [ksplat[kp], jvs[t]]),
                            jnp.bfloat16,
                        )
                        ds[t] = ds[t] + jnp.abs(gj - gi)
                for t in range(GB):
                    dlo, dhi = plsc.unpack(
                        ds[t], format=plsc.PackFormat.INTERLEAVED
                    )
                    acc = acc + wvs[t] * (dlo + dhi)
            return acc

        acc = row

    acc_vm[...] = acc
    pltpu.sync_copy(acc_vm, out_hbm.at[wid])


@functools.cache
def _sc_kernels():
    mesh = plsc.VectorSubcoreMesh(
        core_axis_name="c", subcore_axis_name="s", num_cores=NC, num_subcores=NS
    )
    params = pltpu.CompilerParams(needs_layout_passes=False)
    fused = pl.kernel(
        _fused_body,
        out_type=[
            jax.ShapeDtypeStruct((N * S,), jnp.int32),
            jax.ShapeDtypeStruct((NW, 16), jnp.float32),
        ],
        mesh=mesh,
        scratch_types=[
            pltpu.VMEM((RPW, N), jnp.float32),
            pltpu.VMEM((RPW * S,), jnp.int32),
            pltpu.VMEM((KP2, N), jnp.int32),
            pltpu.VMEM((RC * S,), jnp.int32),
            pltpu.VMEM((16,), jnp.float32),
        ],
        compiler_params=params,
    )
    return fused


def kernel(logits, beliefs, beacon_start_idx):
    fused = _sc_kernels()
    target = beliefs[beliefs.shape[0] - N:]
    ge = beliefs[0:N:2]
    go = beliefs[1:N:2]
    A, ld, bp = _tc_call(logits, target, ge, go)
    _, partials = fused(A, bp)
    loss_box = jnp.sum(partials) / (jnp.float32(beacon_start_idx) * N * N)
    return loss_box + ld[0, 0], A


# speedup vs baseline: 1.0592x; 1.0592x over previous
import functools

import jax
import jax.numpy as jnp
from jax import lax
from jax.experimental import pallas as pl
from jax.experimental.pallas import tpu as pltpu
from jax.experimental.pallas import tpu_sc as plsc

N = 1024
K = 128
TAU = 0.1
S = 128
NC = 2
NS = 16
NW = NC * NS
KC = N // NW
RPW = N // NW
KP = N // NW // 2
RC = 256
HR = N // NC
KP2 = N // NS // 2
GB = 8
MASK_HI = -65536
MASK_LO = 65535
NCH = N // RC
ONE_BITS = 0x3F800001


def _tc_body(logits_ref, target_ref, ge_ref, go_ref, a_ref, ld_ref, bp_ref):
    x = logits_ref[...]
    e = jnp.exp(-jnp.abs(x))
    probs = jnp.where(x >= 0.0, 1.0 / (1.0 + e), e / (1.0 + e))
    pb = lax.bitcast_convert_type(probs, jnp.int32)
    ph = (pb >> 15).astype(jnp.int16)
    plo = (pb & 0x7FFF).astype(jnp.int16)

    def rowsum16(x16):
        s = x16[:, :512] + x16[:, 512:]
        s = s[:, :256] + s[:, 256:]
        s = s[:, :128] + s[:, 128:]
        return jnp.sum(s.astype(jnp.int32), axis=1, keepdims=True)

    lo0 = jnp.zeros((N, 1), jnp.int32)
    hi0 = jnp.full((N, 1), 0x7F01, jnp.int32)

    def it_hi(_, lh):
        lo, hi = lh
        mid = (lo + hi) >> 1
        mid16 = mid.astype(jnp.int16)
        cnt = rowsum16((ph >= mid16).astype(jnp.int16))
        ge = cnt >= K
        return jnp.where(ge, mid, lo), jnp.where(ge, hi, mid)

    th32, _ = lax.fori_loop(0, 15, it_hi, (lo0, hi0))
    th = th32.astype(jnp.int16)

    eq = (ph == th).astype(jnp.int16)
    cgt = rowsum16((ph > th).astype(jnp.int16))

    tlo0 = jnp.zeros((N, 1), jnp.int32)
    thi0 = jnp.full((N, 1), 0x8000, jnp.int32)

    def it_lo(_, lh):
        lo, hi = lh
        mid = (lo + hi) >> 1
        mid16 = mid.astype(jnp.int16)
        cnt = cgt + rowsum16(eq * (plo >= mid16).astype(jnp.int16))
        ge = cnt >= K
        return jnp.where(ge, mid, lo), jnp.where(ge, hi, mid)

    tl, _ = lax.fori_loop(0, 15, it_lo, (tlo0, thi0))
    kth = (th32 << 15) | tl
    A = jnp.where(pb >= kth, probs, 0.0)
    a_ref[...] = A

    we = (A * target_ref[...]) * (1.0 / TAU)
    m = jnp.max(we, axis=1, keepdims=True)
    s = jnp.sum(jnp.exp(we - m), axis=1, keepdims=True)
    me = TAU * (m + jnp.log(s))
    ld_ref[...] = jnp.sum((1.0 - me) ** 2, axis=0, keepdims=True) * (1.0 / N)

    lo = lax.bitcast_convert_type(
        ge_ref[...].astype(jnp.bfloat16), jnp.uint16
    ).astype(jnp.int32)
    hi = lax.bitcast_convert_type(
        go_ref[...].astype(jnp.bfloat16), jnp.uint16
    ).astype(jnp.int32)
    bp_ref[...] = lo | (hi << 16)


_tc_call = pl.pallas_call(
    _tc_body,
    out_shape=[
        jax.ShapeDtypeStruct((N, N), jnp.float32),
        jax.ShapeDtypeStruct((1, 1), jnp.float32),
        jax.ShapeDtypeStruct((N // 2, N), jnp.int32),
    ],
)


def _fused_body(a_hbm, bp_hbm, jw_hbm, out_hbm, a_vm, jwx_vm, bp_vm, jw_vm, acc_vm):
    c = lax.axis_index("c")
    s = lax.axis_index("s")
    wid = c * NS + s
    r0 = wid * RPW

    pltpu.sync_copy(a_hbm.at[pl.ds(r0, RPW)], a_vm)

    iota = lax.iota(jnp.int32, 16)
    vone = jnp.full((16,), 1, jnp.int32)
    vzero = jnp.zeros((16,), jnp.int32)

    @plsc.parallel_loop(0, RPW, unroll=4)
    def erow(r):
        base0 = jnp.full((16,), r * S, jnp.int32)
        lim = base0 + S

        def vloop(v, base):
            a = a_vm[r, pl.ds(v * 16, 16)]
            m = a != 0.0
            ones = jnp.where(m, vone, vzero)
            pos = base + (plsc.cumsum(ones) - ones)
            ok = m & (pos < lim)
            wbits = (plsc.bitcast(a, jnp.int32) + 0x8000) & MASK_HI
            plsc.store_scatter(jwx_vm, [pos], (iota + v * 16) | wbits, mask=ok)
            return base + jnp.full((16,), jnp.sum(ones), jnp.int32)

        lax.fori_loop(0, N // 16, vloop, base0)

    pltpu.sync_copy(jwx_vm, jw_hbm.at[pl.ds(r0 * S, RPW * S)])

    plsc.subcore_barrier()

    pltpu.sync_copy(bp_hbm.at[pl.ds(s * KP2, KP2)], bp_vm)

    ksplat = [jnp.full((16,), kp, jnp.int32) for kp in range(KP2)]
    acc = jnp.zeros((16,), jnp.float32)
    for ch in range(HR // RC):
        row_base = c * HR + ch * RC
        pltpu.sync_copy(jw_hbm.at[pl.ds(row_base * S, RC * S)], jw_vm)

        @plsc.parallel_loop(0, RC, carry=acc, unroll=2)
        def row(r, acc):
            ii = jnp.full((16,), row_base + r, jnp.int32)
            for b in range(S // 16 // GB):
                offs = [r * S + (b * GB + t) * 16 for t in range(GB)]
                jws = [jw_vm[pl.ds(o, 16)] for o in offs]
                jvs = [jw & MASK_LO for jw in jws]
                wvs = [plsc.bitcast(jw & MASK_HI, jnp.float32) for jw in jws]
                ds = [jnp.zeros((32,), jnp.bfloat16) for _ in range(GB)]
                for kp in range(KP2):
                    gi = plsc.bitcast(
                        plsc.load_gather(bp_vm, [ksplat[kp], ii]), jnp.bfloat16
                    )
                    for t in range(GB):
                        gj = plsc.bitcast(
                            plsc.load_gather(bp_vm, [ksplat[kp], jvs[t]]),
                            jnp.bfloat16,
                        )
                        ds[t] = ds[t] + jnp.abs(gj - gi)
                for t in range(GB):
                    dlo, dhi = plsc.unpack(
                        ds[t], format=plsc.PackFormat.INTERLEAVED
                    )
                    acc = acc + wvs[t] * (dlo + dhi)
            return acc

        acc = row

    acc_vm[...] = acc
    pltpu.sync_copy(acc_vm, out_hbm.at[wid])


@functools.cache
def _sc_kernels():
    mesh = plsc.VectorSubcoreMesh(
        core_axis_name="c", subcore_axis_name="s", num_cores=NC, num_subcores=NS
    )
    params = pltpu.CompilerParams(needs_layout_passes=False)
    fused = pl.kernel(
        _fused_body,
        out_type=[
            jax.ShapeDtypeStruct((N * S,), jnp.int32),
            jax.ShapeDtypeStruct((NW, 16), jnp.float32),
        ],
        mesh=mesh,
        scratch_types=[
            pltpu.VMEM((RPW, N), jnp.float32),
            pltpu.VMEM((RPW * S,), jnp.int32),
            pltpu.VMEM((KP2, N), jnp.int32),
            pltpu.VMEM((RC * S,), jnp.int32),
            pltpu.VMEM((16,), jnp.float32),
        ],
        compiler_params=params,
    )
    return fused


def kernel(logits, beliefs, beacon_start_idx):
    fused = _sc_kernels()
    target = beliefs[beliefs.shape[0] - N:]
    ge = beliefs[0:N:2]
    go = beliefs[1:N:2]
    A, ld, bp = _tc_call(logits, target, ge, go)
    _, partials = fused(A, bp)
    loss_box = jnp.sum(partials) / (jnp.float32(beacon_start_idx) * N * N)
    return loss_box + ld[0, 0], A


# speedup vs baseline: 1.0602x; 1.0010x over previous
import functools

import jax
import jax.numpy as jnp
from jax import lax
from jax.experimental import pallas as pl
from jax.experimental.pallas import tpu as pltpu
from jax.experimental.pallas import tpu_sc as plsc

N = 1024
K = 128
TAU = 0.1
S = 128
NC = 2
NS = 16
NW = NC * NS
KC = N // NW
RPW = N // NW
KP = N // NW // 2
RC = 256
HR = N // NC
KP2 = N // NS // 2
GB = 8
MASK_HI = -65536
MASK_LO = 65535
NCH = N // RC
ONE_BITS = 0x3F800001


def _tc_body(logits_ref, target_ref, ge_ref, go_ref, a_ref, ld_ref, bp_ref):
    x = logits_ref[...]
    e = jnp.exp(-jnp.abs(x))
    probs = jnp.where(x >= 0.0, 1.0 / (1.0 + e), e / (1.0 + e))
    pb = lax.bitcast_convert_type(probs, jnp.int32)
    ph = (pb >> 15).astype(jnp.int16)
    plo = (pb & 0x7FFF).astype(jnp.int16)

    def rowsum16(x16):
        s = x16[:, :512] + x16[:, 512:]
        s = s[:, :256] + s[:, 256:]
        s = s[:, :128] + s[:, 128:]
        return jnp.sum(s.astype(jnp.int32), axis=1, keepdims=True)

    lo0 = jnp.zeros((N, 1), jnp.int32)
    hi0 = jnp.full((N, 1), 0x7F01, jnp.int32)

    def it_hi(_, lh):
        lo, hi = lh
        mid = (lo + hi) >> 1
        mid16 = mid.astype(jnp.int16)
        cnt = rowsum16((ph >= mid16).astype(jnp.int16))
        ge = cnt >= K
        return jnp.where(ge, mid, lo), jnp.where(ge, hi, mid)

    th32, _ = lax.fori_loop(0, 15, it_hi, (lo0, hi0))
    th = th32.astype(jnp.int16)

    eq = (ph == th).astype(jnp.int16)
    cgt = rowsum16((ph > th).astype(jnp.int16))

    tlo0 = jnp.zeros((N, 1), jnp.int32)
    thi0 = jnp.full((N, 1), 0x8000, jnp.int32)

    def it_lo(_, lh):
        lo, hi = lh
        mid = (lo + hi) >> 1
        mid16 = mid.astype(jnp.int16)
        cnt = cgt + rowsum16(eq * (plo >= mid16).astype(jnp.int16))
        ge = cnt >= K
        return jnp.where(ge, mid, lo), jnp.where(ge, hi, mid)

    tl, _ = lax.fori_loop(0, 15, it_lo, (tlo0, thi0))
    kth = (th32 << 15) | tl
    A = jnp.where(pb >= kth, probs, 0.0)
    a_ref[...] = A

    we = (A * target_ref[...]) * (1.0 / TAU)
    m = jnp.max(we, axis=1, keepdims=True)
    s = jnp.sum(jnp.exp(we - m), axis=1, keepdims=True)
    me = TAU * (m + jnp.log(s))
    ld_ref[...] = jnp.sum((1.0 - me) ** 2, axis=0, keepdims=True) * (1.0 / N)

    lo = lax.bitcast_convert_type(
        ge_ref[...].astype(jnp.bfloat16), jnp.uint16
    ).astype(jnp.int32)
    hi = lax.bitcast_convert_type(
        go_ref[...].astype(jnp.bfloat16), jnp.uint16
    ).astype(jnp.int32)
    bp_ref[...] = lo | (hi << 16)


_tc_call = pl.pallas_call(
    _tc_body,
    out_shape=[
        jax.ShapeDtypeStruct((N, N), jnp.float32),
        jax.ShapeDtypeStruct((1, 1), jnp.float32),
        jax.ShapeDtypeStruct((N // 2, N), jnp.int32),
    ],
)


def _fused_body(a_hbm, bp_hbm, jw_hbm, out_hbm, a_vm, jwx_vm, bp_vm, jw_vm, acc_vm):
    c = lax.axis_index("c")
    s = lax.axis_index("s")
    wid = c * NS + s
    r0 = wid * RPW

    pltpu.sync_copy(a_hbm.at[pl.ds(r0, RPW)], a_vm)

    iota = lax.iota(jnp.int32, 16)
    vone = jnp.full((16,), 1, jnp.int32)
    vzero = jnp.zeros((16,), jnp.int32)

    @plsc.parallel_loop(0, RPW, unroll=2)
    def erow(r):
        base0 = jnp.full((16,), r * S, jnp.int32)
        lim = base0 + S

        def vloop(v, base):
            a = a_vm[r, pl.ds(v * 16, 16)]
            m = a != 0.0
            ones = jnp.where(m, vone, vzero)
            pos = base + (plsc.cumsum(ones) - ones)
            ok = m & (pos < lim)
            wbits = (plsc.bitcast(a, jnp.int32) + 0x8000) & MASK_HI
            plsc.store_scatter(jwx_vm, [pos], (iota + v * 16) | wbits, mask=ok)
            return base + jnp.full((16,), jnp.sum(ones), jnp.int32)

        lax.fori_loop(0, N // 16, vloop, base0)

    pltpu.sync_copy(jwx_vm, jw_hbm.at[pl.ds(r0 * S, RPW * S)])

    plsc.subcore_barrier()

    pltpu.sync_copy(bp_hbm.at[pl.ds(s * KP2, KP2)], bp_vm)

    ksplat = [jnp.full((16,), kp, jnp.int32) for kp in range(KP2)]
    acc = jnp.zeros((16,), jnp.float32)
    for ch in range(HR // RC):
        row_base = c * HR + ch * RC
        pltpu.sync_copy(jw_hbm.at[pl.ds(row_base * S, RC * S)], jw_vm)

        @plsc.parallel_loop(0, RC, carry=acc, unroll=2)
        def row(r, acc):
            ii = jnp.full((16,), row_base + r, jnp.int32)
            for b in range(S // 16 // GB):
                offs = [r * S + (b * GB + t) * 16 for t in range(GB)]
                jws = [jw_vm[pl.ds(o, 16)] for o in offs]
                jvs = [jw & MASK_LO for jw in jws]
                wvs = [plsc.bitcast(jw & MASK_HI, jnp.float32) for jw in jws]
                ds = [jnp.zeros((32,), jnp.bfloat16) for _ in range(GB)]
                for kp in range(KP2):
                    gi = plsc.bitcast(
                        plsc.load_gather(bp_vm, [ksplat[kp], ii]), jnp.bfloat16
                    )
                    for t in range(GB):
                        gj = plsc.bitcast(
                            plsc.load_gather(bp_vm, [ksplat[kp], jvs[t]]),
                            jnp.bfloat16,
                        )
                        ds[t] = ds[t] + jnp.abs(gj - gi)
                for t in range(GB):
                    dlo, dhi = plsc.unpack(
                        ds[t], format=plsc.PackFormat.INTERLEAVED
                    )
                    acc = acc + wvs[t] * (dlo + dhi)
            return acc

        acc = row

    acc_vm[...] = acc
    pltpu.sync_copy(acc_vm, out_hbm.at[wid])


@functools.cache
def _sc_kernels():
    mesh = plsc.VectorSubcoreMesh(
        core_axis_name="c", subcore_axis_name="s", num_cores=NC, num_subcores=NS
    )
    params = pltpu.CompilerParams(needs_layout_passes=False)
    fused = pl.kernel(
        _fused_body,
        out_type=[
            jax.ShapeDtypeStruct((N * S,), jnp.int32),
            jax.ShapeDtypeStruct((NW, 16), jnp.float32),
        ],
        mesh=mesh,
        scratch_types=[
            pltpu.VMEM((RPW, N), jnp.float32),
            pltpu.VMEM((RPW * S,), jnp.int32),
            pltpu.VMEM((KP2, N), jnp.int32),
            pltpu.VMEM((RC * S,), jnp.int32),
            pltpu.VMEM((16,), jnp.float32),
        ],
        compiler_params=params,
    )
    return fused


def kernel(logits, beliefs, beacon_start_idx):
    fused = _sc_kernels()
    target = beliefs[beliefs.shape[0] - N:]
    ge = beliefs[0:N:2]
    go = beliefs[1:N:2]
    A, ld, bp = _tc_call(logits, target, ge, go)
    _, partials = fused(A, bp)
    loss_box = jnp.sum(partials) / (jnp.float32(beacon_start_idx) * N * N)
    return loss_box + ld[0, 0], A
